# baseline (speedup 1.0000x reference)
import functools

import jax
import jax.numpy as jnp
from jax.experimental import pallas as pl
from jax.experimental.pallas import tpu as pltpu

_NEG_INF = float("-inf")


def _i2pool_kernel(x_ref, g_ref, o_ref):
    v = x_ref[0] * g_ref[0]
    w = v.shape[-1]
    s = 1
    while s < w:
        shifted = jnp.concatenate(
            [jnp.full(v.shape[:-1] + (s,), _NEG_INF, v.dtype), v[..., :-s]],
            axis=-1,
        )
        v = jnp.maximum(v, shifted)
        s *= 2
    o_ref[0] = v


@jax.jit
def kernel(x, guide):
    b, c, h, w = x.shape
    cb = 32
    grid = (b, c // cb)
    return pl.pallas_call(
        _i2pool_kernel,
        grid=grid,
        in_specs=[
            pl.BlockSpec((1, cb, h, w), lambda i, j: (i, j, 0, 0)),
            pl.BlockSpec((1, 1, h, w), lambda i, j: (i, 0, 0, 0)),
        ],
        out_specs=pl.BlockSpec((1, cb, h, w), lambda i, j: (i, j, 0, 0)),
        out_shape=jax.ShapeDtypeStruct(x.shape, x.dtype),
        compiler_params=pltpu.CompilerParams(
            dimension_semantics=("parallel", "parallel"),
        ),
    )(x, guide)


# speedup vs baseline: 6.7894x; 6.7894x over previous
import jax
import jax.numpy as jnp
from jax.experimental import pallas as pl
from jax.experimental.pallas import tpu as pltpu


def _i2pool_body(x_ref, g_ref, o_ref):
    hb = x_ref.shape[1]
    for h in range(hb):
        v = x_ref[0, h] * g_ref[0, 0, :, h][:, None]
        w = v.shape[0]
        s = 1
        while s < w:
            v = jnp.concatenate([v[:s], jnp.maximum(v[s:], v[:-s])], axis=0)
            s *= 2
        o_ref[0, h] = v


@jax.jit
def kernel(x, guide):
    b, c, h, w = x.shape
    xt = jnp.transpose(x, (0, 2, 3, 1))
    hb = 48 if h % 48 == 0 else h
    grid = (b, h // hb)
    gt = jnp.transpose(guide[:, 0].reshape(b, h // hb, hb, w), (0, 1, 3, 2))
    out = pl.pallas_call(
        _i2pool_body,
        grid=grid,
        in_specs=[
            pl.BlockSpec((1, hb, w, c), lambda i, j: (i, j, 0, 0)),
            pl.BlockSpec((1, 1, w, hb), lambda i, j: (i, j, 0, 0)),
        ],
        out_specs=pl.BlockSpec((1, hb, w, c), lambda i, j: (i, j, 0, 0)),
        out_shape=jax.ShapeDtypeStruct((b, h, w, c), x.dtype),
        compiler_params=pltpu.CompilerParams(
            dimension_semantics=("parallel", "parallel"),
        ),
    )(xt, gt)
    return jnp.transpose(out, (0, 3, 1, 2))


# speedup vs baseline: 7.1961x; 1.0599x over previous
import jax
import jax.numpy as jnp
from jax.experimental import pallas as pl
from jax.experimental.pallas import tpu as pltpu


def _i2pool_body(x_ref, g_ref, o_ref):
    hb = x_ref.shape[1]
    for h in range(hb):
        v = x_ref[0, h] * g_ref[0, 0, :, h][:, None]
        w = v.shape[0]
        s = 1
        while s < w:
            v = jnp.concatenate([v[:s], jnp.maximum(v[s:], v[:-s])], axis=0)
            s *= 2
        o_ref[0, h] = v


@jax.jit
def kernel(x, guide):
    b, c, h, w = x.shape
    xt = jnp.transpose(x, (0, 2, 3, 1))
    hb = 96 if h % 96 == 0 else h
    grid = (b, h // hb)
    gt = jnp.transpose(guide[:, 0].reshape(b, h // hb, hb, w), (0, 1, 3, 2))
    out = pl.pallas_call(
        _i2pool_body,
        grid=grid,
        in_specs=[
            pl.BlockSpec((1, hb, w, c), lambda i, j: (i, j, 0, 0)),
            pl.BlockSpec((1, 1, w, hb), lambda i, j: (i, j, 0, 0)),
        ],
        out_specs=pl.BlockSpec((1, hb, w, c), lambda i, j: (i, j, 0, 0)),
        out_shape=jax.ShapeDtypeStruct((b, h, w, c), x.dtype),
        compiler_params=pltpu.CompilerParams(
            dimension_semantics=("parallel", "parallel"),
        ),
    )(xt, gt)
    return jnp.transpose(out, (0, 3, 1, 2))
